# Initial kernel scaffold; baseline (speedup 1.0000x reference)
import jax
import jax.numpy as jnp
from jax.experimental import pallas as pl


def kernel(h, coord, edge_index, coord_diff, edge_attr, W1, b1, W2, b2, W3):
    raise NotImplementedError("write your pallas kernel here")


# speedup vs baseline: 2.8590x; 2.8590x over previous
import functools

import jax
import jax.numpy as jnp
from jax import lax
from jax.experimental import pallas as pl
from jax.experimental.pallas import tpu as pltpu
from jax.experimental.pallas import tpu_sc as plsc

NC, NS = 2, 16
NW = NC * NS
GC = 80
SB = 80
NORM = 100.0


def _pre_body(h_ref, w1a_ref, w1b_ref, t_ref):
    h = h_ref[...]
    t_ref[0, :, :] = jnp.dot(h, w1a_ref[...], preferred_element_type=jnp.float32)
    t_ref[1, :, :] = jnp.dot(h, w1b_ref[...], preferred_element_type=jnp.float32)


def _precompute(h, w1a, w1b):
    n, hid = h.shape
    bn = 2000
    return pl.pallas_call(
        _pre_body,
        grid=(n // bn,),
        in_specs=[
            pl.BlockSpec((bn, hid), lambda i: (i, 0)),
            pl.BlockSpec((hid, hid), lambda i: (0, 0)),
            pl.BlockSpec((hid, hid), lambda i: (0, 0)),
        ],
        out_specs=pl.BlockSpec((2, bn, hid), lambda i: (0, i, 0)),
        out_shape=jax.ShapeDtypeStruct((2, n, hid), jnp.float32),
    )(h, w1a, w1b)


def _gather(table, gidx):
    tot, hid = gidx.shape[0], table.shape[1]
    per_w = tot // NW
    nch = per_w // GC
    mesh = plsc.VectorSubcoreMesh(
        core_axis_name="c", subcore_axis_name="s", num_cores=NC, num_subcores=NS)

    @functools.partial(
        pl.kernel, mesh=mesh,
        out_type=jax.ShapeDtypeStruct((tot, hid), jnp.float32),
        scratch_types=[
            pltpu.VMEM((GC,), jnp.int32),
            pltpu.VMEM((GC, hid), jnp.float32),
            pltpu.SemaphoreType.DMA,
        ],
    )
    def k(tbl_hbm, idx_hbm, out_hbm, idx_v, rows_v, sem):
        wid = lax.axis_index("s") * NC + lax.axis_index("c")
        base = wid * per_w

        def chunk(j, carry):
            off = base + j * GC
            pltpu.sync_copy(idx_hbm.at[pl.ds(off, GC)], idx_v)
            pltpu.async_copy(tbl_hbm.at[idx_v], rows_v, sem).wait()
            pltpu.sync_copy(rows_v, out_hbm.at[pl.ds(off, GC)])
            return carry

        lax.fori_loop(0, nch, chunk, 0)

    return k(table, gidx)


def _mlp_body(g_ref, ea_ref, cd_ref, w1c_ref, b1_ref, w2_ref, b2_ref, w3_ref,
              t0_ref, t1_ref, t2_ref):
    g = (g_ref[0, :, :] + g_ref[1, :, :] + b1_ref[...]
         + jnp.dot(ea_ref[...], w1c_ref[...], preferred_element_type=jnp.float32))
    x1 = g * jax.nn.sigmoid(g)
    x2 = jnp.dot(x1, w2_ref[...], preferred_element_type=jnp.float32) + b2_ref[...]
    x2 = x2 * jax.nn.sigmoid(x2)
    out_t = lax.dot_general(w3_ref[...], x2, (((1,), (1,)), ((), ())),
                            preferred_element_type=jnp.float32)
    cd = cd_ref[...]
    t0_ref[...] = cd[0:1, :] * out_t
    t1_ref[...] = cd[1:2, :] * out_t
    t2_ref[...] = cd[2:3, :] * out_t


def _mlp(g2, ea, cdt, w1c, b1, w2, b2, w3row):
    _, e, hid = g2.shape
    be = 1280
    ed = ea.shape[1]
    row_sds = jax.ShapeDtypeStruct((1, e), jnp.float32)
    return pl.pallas_call(
        _mlp_body,
        grid=(e // be,),
        in_specs=[
            pl.BlockSpec((2, be, hid), lambda i: (0, i, 0)),
            pl.BlockSpec((be, ed), lambda i: (i, 0)),
            pl.BlockSpec((3, be), lambda i: (0, i)),
            pl.BlockSpec((ed, hid), lambda i: (0, 0)),
            pl.BlockSpec((1, hid), lambda i: (0, 0)),
            pl.BlockSpec((hid, hid), lambda i: (0, 0)),
            pl.BlockSpec((1, hid), lambda i: (0, 0)),
            pl.BlockSpec((1, hid), lambda i: (0, 0)),
        ],
        out_specs=[
            pl.BlockSpec((1, be), lambda i: (0, i)),
            pl.BlockSpec((1, be), lambda i: (0, i)),
            pl.BlockSpec((1, be), lambda i: (0, i)),
        ],
        out_shape=[row_sds, row_sds, row_sds],
    )(g2, ea, cdt, w1c, b1, w2, b2, w3row)


def _scatter(tr0, tr1, tr2, row_r, zeros_n, n):
    e = tr0.shape[0]
    ew = e // NW
    nch = ew // SB
    rpt = 1000
    ntc = n // rpt
    mesh = plsc.VectorSubcoreMesh(
        core_axis_name="c", subcore_axis_name="s", num_cores=NC, num_subcores=NS)
    part = jax.ShapeDtypeStruct((n,), jnp.float32)

    @functools.partial(
        pl.kernel, mesh=mesh,
        out_type=[part] * 6,
        scratch_types=[
            pltpu.VMEM((ew // SB, SB), jnp.int32),
            pltpu.VMEM((ew,), jnp.float32),
            pltpu.VMEM((ew,), jnp.float32),
            pltpu.VMEM((ew,), jnp.float32),
            pltpu.VMEM_SHARED((n,), jnp.float32),
            pltpu.VMEM_SHARED((n,), jnp.float32),
            pltpu.VMEM_SHARED((n,), jnp.float32),
        ],
        compiler_params=pltpu.CompilerParams(use_tc_tiling_on_sc=False),
    )
    def k(tr0_hbm, tr1_hbm, tr2_hbm, rowr_hbm, z_hbm,
          o00, o01, o02, o10, o11, o12,
          idx_v, t0_v, t1_v, t2_v, a0, a1, a2):
        cid = lax.axis_index("c")
        sid = lax.axis_index("s")
        wid = sid * NC + cid

        @pl.when(sid == 0)
        def _():
            pltpu.sync_copy(z_hbm, a0)
            pltpu.sync_copy(z_hbm, a1)
            pltpu.sync_copy(z_hbm, a2)

        plsc.subcore_barrier()

        sl_in = pl.ds(wid * ew, ew)
        pltpu.sync_copy(rowr_hbm.at[wid], idx_v)
        pltpu.sync_copy(tr0_hbm.at[sl_in], t0_v)
        pltpu.sync_copy(tr1_hbm.at[sl_in], t1_v)
        pltpu.sync_copy(tr2_hbm.at[sl_in], t2_v)

        def chunk(j, carry):
            sl = pl.ds(j * SB, SB)
            pltpu.sync_copy(t0_v.at[sl], a0.at[idx_v.at[j]], add=True)
            pltpu.sync_copy(t1_v.at[sl], a1.at[idx_v.at[j]], add=True)
            pltpu.sync_copy(t2_v.at[sl], a2.at[idx_v.at[j]], add=True)
            return carry

        lax.fori_loop(0, nch, chunk, 0)

        plsc.subcore_barrier()

        @pl.when(sid < ntc)
        def _():
            sl = pl.ds(sid * rpt, rpt)

            @pl.when(cid == 0)
            def _():
                pltpu.sync_copy(a0.at[sl], o00.at[sl])
                pltpu.sync_copy(a1.at[sl], o01.at[sl])
                pltpu.sync_copy(a2.at[sl], o02.at[sl])

            @pl.when(cid == 1)
            def _():
                pltpu.sync_copy(a0.at[sl], o10.at[sl])
                pltpu.sync_copy(a1.at[sl], o11.at[sl])
                pltpu.sync_copy(a2.at[sl], o12.at[sl])

    return k(tr0, tr1, tr2, row_r, zeros_n)


def kernel(h, coord, edge_index, coord_diff, edge_attr, W1, b1, W2, b2, W3):
    n, hid = h.shape
    e = edge_index.shape[1]
    row = edge_index[0]
    col = edge_index[1]

    t = _precompute(h, W1[:hid], W1[hid:2 * hid])
    gidx = jnp.concatenate([row, col + n])
    g = _gather(t.reshape(2 * n, hid), gidx)

    cdt = jnp.transpose(coord_diff)
    tr0, tr1, tr2 = _mlp(g.reshape(2, e, hid), edge_attr, cdt,
                         W1[2 * hid:], b1.reshape(1, hid), W2,
                         b2.reshape(1, hid), W3.reshape(1, hid))

    row_r = row.reshape(NW, e // NW // SB, SB)
    zeros_n = jnp.zeros((n,), jnp.float32)
    parts = _scatter(tr0.reshape(e), tr1.reshape(e), tr2.reshape(e),
                     row_r, zeros_n, n)

    agg = jnp.stack([parts[0] + parts[3],
                     parts[1] + parts[4],
                     parts[2] + parts[5]], axis=1) / NORM
    return coord + agg


# speedup vs baseline: 3.7680x; 1.3179x over previous
import functools

import jax
import jax.numpy as jnp
from jax import lax
from jax.experimental import pallas as pl
from jax.experimental.pallas import tpu as pltpu
from jax.experimental.pallas import tpu_sc as plsc

NC, NS = 2, 16
NW = NC * NS
GC = 80
SB = 80
NORM = 100.0


def _pre_body(h_ref, w1a_ref, w1b_ref, t_ref):
    h = h_ref[...]
    t_ref[0, :, :] = jnp.dot(h, w1a_ref[...], preferred_element_type=jnp.float32)
    t_ref[1, :, :] = jnp.dot(h, w1b_ref[...], preferred_element_type=jnp.float32)


def _precompute(h, w1a, w1b):
    n, hid = h.shape
    bn = 2000
    return pl.pallas_call(
        _pre_body,
        grid=(n // bn,),
        in_specs=[
            pl.BlockSpec((bn, hid), lambda i: (i, 0)),
            pl.BlockSpec((hid, hid), lambda i: (0, 0)),
            pl.BlockSpec((hid, hid), lambda i: (0, 0)),
        ],
        out_specs=pl.BlockSpec((2, bn, hid), lambda i: (0, i, 0)),
        out_shape=jax.ShapeDtypeStruct((2, n, hid), jnp.float32),
    )(h, w1a, w1b)


def _gather(table, gidx):
    tot, hid = gidx.shape[0], table.shape[1]
    per_w = tot // NW
    nch = per_w // GC
    mesh = plsc.VectorSubcoreMesh(
        core_axis_name="c", subcore_axis_name="s", num_cores=NC, num_subcores=NS)

    npair = nch // 2

    @functools.partial(
        pl.kernel, mesh=mesh,
        out_type=jax.ShapeDtypeStruct((tot, hid), jnp.float32),
        scratch_types=[
            pltpu.VMEM((per_w,), jnp.int32),
            pltpu.VMEM((GC, hid), jnp.float32),
            pltpu.VMEM((GC, hid), jnp.float32),
            pltpu.SemaphoreType.DMA,
            pltpu.SemaphoreType.DMA,
            pltpu.SemaphoreType.DMA,
            pltpu.SemaphoreType.DMA,
        ],
        compiler_params=pltpu.CompilerParams(use_tc_tiling_on_sc=False),
    )
    def k(tbl_hbm, idx_hbm, out_hbm, idx_v, rows_a, rows_b, sga, sgb, soa, sob):
        wid = lax.axis_index("s") * NC + lax.axis_index("c")
        base = wid * per_w
        pltpu.sync_copy(idx_hbm.at[pl.ds(base, per_w)], idx_v)

        def pair(k_, carry):
            ja = 2 * k_
            jb = 2 * k_ + 1

            @pl.when(k_ > 0)
            def _():
                pltpu.make_async_copy(rows_a, out_hbm.at[pl.ds(base, GC)], soa).wait()
                pltpu.make_async_copy(rows_b, out_hbm.at[pl.ds(base, GC)], sob).wait()

            ga = pltpu.async_copy(
                tbl_hbm.at[idx_v.at[pl.ds(ja * GC, GC)]], rows_a, sga)
            gb = pltpu.async_copy(
                tbl_hbm.at[idx_v.at[pl.ds(jb * GC, GC)]], rows_b, sgb)
            ga.wait()
            pltpu.async_copy(rows_a, out_hbm.at[pl.ds(base + ja * GC, GC)], soa)
            gb.wait()
            pltpu.async_copy(rows_b, out_hbm.at[pl.ds(base + jb * GC, GC)], sob)
            return carry

        lax.fori_loop(0, npair, pair, 0)
        pltpu.make_async_copy(rows_a, out_hbm.at[pl.ds(base, GC)], soa).wait()
        pltpu.make_async_copy(rows_b, out_hbm.at[pl.ds(base, GC)], sob).wait()

    return k(table, gidx)


def _mlp_body(g_ref, ea_ref, cd_ref, w1c_ref, b1_ref, w2_ref, b2_ref, w3_ref,
              t0_ref, t1_ref, t2_ref):
    g = (g_ref[0, :, :] + g_ref[1, :, :] + b1_ref[...]
         + jnp.dot(ea_ref[...], w1c_ref[...], preferred_element_type=jnp.float32))
    x1 = g * jax.nn.sigmoid(g)
    x2 = jnp.dot(x1, w2_ref[...], preferred_element_type=jnp.float32) + b2_ref[...]
    x2 = x2 * jax.nn.sigmoid(x2)
    out_t = lax.dot_general(w3_ref[...], x2, (((1,), (1,)), ((), ())),
                            preferred_element_type=jnp.float32)
    cd = cd_ref[...]
    t0_ref[...] = cd[0:1, :] * out_t
    t1_ref[...] = cd[1:2, :] * out_t
    t2_ref[...] = cd[2:3, :] * out_t


def _mlp(g2, ea, cdt, w1c, b1, w2, b2, w3row):
    _, e, hid = g2.shape
    be = 1280
    ed = ea.shape[1]
    row_sds = jax.ShapeDtypeStruct((1, e), jnp.float32)
    return pl.pallas_call(
        _mlp_body,
        grid=(e // be,),
        in_specs=[
            pl.BlockSpec((2, be, hid), lambda i: (0, i, 0)),
            pl.BlockSpec((be, ed), lambda i: (i, 0)),
            pl.BlockSpec((3, be), lambda i: (0, i)),
            pl.BlockSpec((ed, hid), lambda i: (0, 0)),
            pl.BlockSpec((1, hid), lambda i: (0, 0)),
            pl.BlockSpec((hid, hid), lambda i: (0, 0)),
            pl.BlockSpec((1, hid), lambda i: (0, 0)),
            pl.BlockSpec((1, hid), lambda i: (0, 0)),
        ],
        out_specs=[
            pl.BlockSpec((1, be), lambda i: (0, i)),
            pl.BlockSpec((1, be), lambda i: (0, i)),
            pl.BlockSpec((1, be), lambda i: (0, i)),
        ],
        out_shape=[row_sds, row_sds, row_sds],
    )(g2, ea, cdt, w1c, b1, w2, b2, w3row)


def _scatter(tr0, tr1, tr2, row_r, zeros_n, n):
    e = tr0.shape[0]
    ew = e // NW
    nch = ew // SB
    rpt = 1000
    ntc = n // rpt
    mesh = plsc.VectorSubcoreMesh(
        core_axis_name="c", subcore_axis_name="s", num_cores=NC, num_subcores=NS)
    part = jax.ShapeDtypeStruct((n,), jnp.float32)

    @functools.partial(
        pl.kernel, mesh=mesh,
        out_type=[part] * 6,
        scratch_types=[
            pltpu.VMEM((ew // SB, SB), jnp.int32),
            pltpu.VMEM((ew,), jnp.float32),
            pltpu.VMEM((ew,), jnp.float32),
            pltpu.VMEM((ew,), jnp.float32),
            pltpu.VMEM_SHARED((n,), jnp.float32),
            pltpu.VMEM_SHARED((n,), jnp.float32),
            pltpu.VMEM_SHARED((n,), jnp.float32),
        ],
        compiler_params=pltpu.CompilerParams(use_tc_tiling_on_sc=False),
    )
    def k(tr0_hbm, tr1_hbm, tr2_hbm, rowr_hbm, z_hbm,
          o00, o01, o02, o10, o11, o12,
          idx_v, t0_v, t1_v, t2_v, a0, a1, a2):
        cid = lax.axis_index("c")
        sid = lax.axis_index("s")
        wid = sid * NC + cid

        @pl.when(sid == 0)
        def _():
            pltpu.sync_copy(z_hbm, a0)
            pltpu.sync_copy(z_hbm, a1)
            pltpu.sync_copy(z_hbm, a2)

        plsc.subcore_barrier()

        sl_in = pl.ds(wid * ew, ew)
        pltpu.sync_copy(rowr_hbm.at[wid], idx_v)
        pltpu.sync_copy(tr0_hbm.at[sl_in], t0_v)
        pltpu.sync_copy(tr1_hbm.at[sl_in], t1_v)
        pltpu.sync_copy(tr2_hbm.at[sl_in], t2_v)

        def chunk(j, carry):
            sl = pl.ds(j * SB, SB)
            pltpu.sync_copy(t0_v.at[sl], a0.at[idx_v.at[j]], add=True)
            pltpu.sync_copy(t1_v.at[sl], a1.at[idx_v.at[j]], add=True)
            pltpu.sync_copy(t2_v.at[sl], a2.at[idx_v.at[j]], add=True)
            return carry

        lax.fori_loop(0, nch, chunk, 0)

        plsc.subcore_barrier()

        @pl.when(sid < ntc)
        def _():
            sl = pl.ds(sid * rpt, rpt)

            @pl.when(cid == 0)
            def _():
                pltpu.sync_copy(a0.at[sl], o00.at[sl])
                pltpu.sync_copy(a1.at[sl], o01.at[sl])
                pltpu.sync_copy(a2.at[sl], o02.at[sl])

            @pl.when(cid == 1)
            def _():
                pltpu.sync_copy(a0.at[sl], o10.at[sl])
                pltpu.sync_copy(a1.at[sl], o11.at[sl])
                pltpu.sync_copy(a2.at[sl], o12.at[sl])

    return k(tr0, tr1, tr2, row_r, zeros_n)


def kernel(h, coord, edge_index, coord_diff, edge_attr, W1, b1, W2, b2, W3):
    n, hid = h.shape
    e = edge_index.shape[1]
    row = edge_index[0]
    col = edge_index[1]

    t = _precompute(h, W1[:hid], W1[hid:2 * hid])
    gidx = jnp.concatenate([row, col + n])
    g = _gather(t.reshape(2 * n, hid), gidx)

    cdt = jnp.transpose(coord_diff)
    tr0, tr1, tr2 = _mlp(g.reshape(2, e, hid), edge_attr, cdt,
                         W1[2 * hid:], b1.reshape(1, hid), W2,
                         b2.reshape(1, hid), W3.reshape(1, hid))

    row_r = row.reshape(NW, e // NW // SB, SB)
    zeros_n = jnp.zeros((n,), jnp.float32)
    parts = _scatter(tr0.reshape(e), tr1.reshape(e), tr2.reshape(e),
                     row_r, zeros_n, n)

    agg = jnp.stack([parts[0] + parts[3],
                     parts[1] + parts[4],
                     parts[2] + parts[5]], axis=1) / NORM
    return coord + agg
